# baseline (speedup 1.0000x reference)
import functools

import jax
import jax.numpy as jnp
from jax.experimental import pallas as pl
from jax.experimental.pallas import tpu as pltpu

N_NODES = 50000
LIN = 64
OUT = 64
HEADS = 4
SEM_HEADS = 4

_NB = 2048
_NPAD = ((N_NODES + _NB - 1) // _NB) * _NB


def _lrelu(t):
    return jnp.where(t >= 0, t, 0.01 * t)


def _fuse_body(pf, cf, tf, df, wn, bn, wb, bb, wt, bt, wd, bd, w1, b1, o_ref):
    un = _lrelu(jnp.dot(pf[...], wn[...], preferred_element_type=jnp.float32) + bn[...])
    ub = _lrelu(jnp.dot(cf[...], wb[...], preferred_element_type=jnp.float32) + bb[...])
    ut = _lrelu(jnp.dot(tf[...], wt[...], preferred_element_type=jnp.float32) + bt[...])
    ud = _lrelu(jnp.dot(df[...], wd[...], preferred_element_type=jnp.float32) + bd[...])
    x = jnp.concatenate([un, ub, ut, ud], axis=1)
    o_ref[...] = _lrelu(jnp.dot(x, w1[...], preferred_element_type=jnp.float32) + b1[...])


def _fuse(pf, cf, tf, df, p):
    grid = (_NPAD // _NB,)
    def nb(d):
        return pl.BlockSpec((_NB, d), lambda i: (i, 0))
    full = lambda a: pl.BlockSpec(a.shape, lambda i: (0,) * a.ndim)
    args = [
        pf, cf, tf, df,
        p['in_num']['W'], p['in_num']['b'],
        p['in_bool']['W'], p['in_bool']['b'],
        p['in_tweet']['W'], p['in_tweet']['b'],
        p['in_des']['W'], p['in_des']['b'],
        p['lin1']['W'], p['lin1']['b'],
    ]
    specs = [nb(pf.shape[1]), nb(cf.shape[1]), nb(tf.shape[1]), nb(df.shape[1])] + [full(a) for a in args[4:]]
    out = pl.pallas_call(
        _fuse_body,
        grid=grid,
        in_specs=specs,
        out_specs=pl.BlockSpec((_NB, LIN), lambda i: (i, 0)),
        out_shape=jax.ShapeDtypeStruct((_NPAD, LIN), jnp.float32),
    )(*[jnp.pad(a, ((0, _NPAD - N_NODES), (0, 0))) if a.shape[0] == N_NODES else a for a in args])
    return out[:N_NODES]


def _lin(p, x):
    y = x @ p['W']
    if 'b' in p:
        y = y + p['b']
    return y


def _transformer_conv(x, edge_index, tp):
    N = x.shape[0]
    src = edge_index[0]
    dst = edge_index[1]
    q = _lin(tp['q'], x).reshape(N, HEADS, OUT)
    k = _lin(tp['k'], x).reshape(N, HEADS, OUT)
    v = _lin(tp['v'], x).reshape(N, HEADS, OUT)
    alpha = (q[dst] * k[src]).sum(-1) / jnp.sqrt(float(OUT))
    amax = jax.ops.segment_max(alpha, dst, num_segments=N)
    amax = jnp.where(jnp.isfinite(amax), amax, 0.0)
    ex = jnp.exp(alpha - amax[dst])
    denom = jax.ops.segment_sum(ex, dst, num_segments=N)
    w = ex / (denom[dst] + 1e-16)
    out = jax.ops.segment_sum(v[src] * w[:, :, None], dst, num_segments=N)
    out = out.mean(axis=1)
    return out + _lin(tp['s'], x)


def _rgt_layer(x, edges, lp):
    embs = []
    for e in range(2):
        u = _transformer_conv(x, edges[e], lp['tc%d' % e])
        a = jax.nn.sigmoid(_lin(lp['gate'], jnp.concatenate([u, x], axis=1)))
        embs.append(jnp.tanh(u) * a)
    z = jnp.stack(embs, axis=1)
    out = jnp.zeros((x.shape[0], OUT), jnp.float32)
    for h in range(SEM_HEADS):
        hp = lp['sem'][h]
        w = _lin(hp['l2'], jnp.tanh(_lin(hp['l1'], z)))
        w = w.mean(axis=0)
        beta = jax.nn.softmax(w, axis=0)
        out = out + (beta[None, :, :] * z).sum(axis=1)
    return out / float(SEM_HEADS)


def kernel(prop_features, cat_features, tweet_features, des_features, following_edge_index, follows_edge_index, params):
    x = _fuse(prop_features, cat_features, tweet_features, des_features, params)
    edges = [following_edge_index, follows_edge_index]
    x = _lrelu(_rgt_layer(x, edges, params['rgt0']))
    x = _lrelu(_rgt_layer(x, edges, params['rgt1']))
    x = _lrelu(_lin(params['out1'], x))
    return _lin(params['out2'], x)


# speedup vs baseline: 9.0480x; 9.0480x over previous
import jax
import jax.numpy as jnp
from jax import lax
from jax.experimental import pallas as pl
from jax.experimental.pallas import tpu as pltpu
from jax.experimental.pallas import tpu_sc as plsc

N = 50000
NPAD = 51200
E = 800000
EPAD = 802816
C = 128
NW = 32
PW = EPAD // NW
NCH = PW // C
PT = EPAD // 16
NCHA = PT // C
HEADS = 4

_NB = 2048
_GRID = NPAD // _NB

_mesh = plsc.VectorSubcoreMesh(core_axis_name="c", subcore_axis_name="s",
                               num_cores=2, num_subcores=16)


def _lrelu(t):
    return jnp.where(t >= 0, t, 0.01 * t)


def _lanes():
    return lax.broadcasted_iota(jnp.int32, (16,), 0)


CA = 64
NCHQ = PW // CA


def _alpha_body(tok_h, dst_h, src_h, qs_h, k_h, aout_h,
                dstb, srcb, qbuf, kbuf, abufT):
    c = lax.axis_index("c")
    s = lax.axis_index("s")
    w = c * 16 + s
    lane = _lanes()
    lmasks = [jnp.where(lane == t, 1.0, 0.0) for t in range(16)]
    rots = [(lane + r) % 16 for r in (8, 4, 2, 1)]
    base = w * PW

    def _chunk(i, carry):
        off = base + i * CA
        pltpu.sync_copy(dst_h.at[pl.ds(off, CA)], dstb.at[0])
        pltpu.sync_copy(src_h.at[pl.ds(off, CA)], srcb.at[0])
        pltpu.sync_copy(qs_h.at[dstb.at[0]], qbuf)
        pltpu.sync_copy(k_h.at[srcb.at[0]], kbuf)
        for g in range(CA // 16):
            cols = [jnp.zeros((16,), jnp.float32) for _ in range(HEADS)]
            for t in range(16):
                e = g * 16 + t
                for h in range(HEADS):
                    acc = qbuf[e, pl.ds(h * 64, 16)] * kbuf[e, pl.ds(h * 64, 16)]
                    for j in range(1, 4):
                        acc = acc + (qbuf[e, pl.ds(h * 64 + j * 16, 16)]
                                     * kbuf[e, pl.ds(h * 64 + j * 16, 16)])
                    for ridx in rots:
                        acc = acc + jnp.take_along_axis(acc, ridx, axis=0)
                    cols[h] = cols[h] + acc * lmasks[t]
            for h in range(HEADS):
                abufT[h, pl.ds(g * 16, 16)] = cols[h]
        for h in range(HEADS):
            pltpu.sync_copy(abufT.at[h], aout_h.at[pl.ds(h * EPAD + off, CA)])
        return carry
    lax.fori_loop(0, NCHQ, _chunk, 0)


_alpha_call = pl.kernel(
    _alpha_body,
    out_type=jax.ShapeDtypeStruct((4 * EPAD,), jnp.float32),
    mesh=_mesh,
    scratch_types=[
        pltpu.VMEM((1, CA), jnp.int32),
        pltpu.VMEM((1, CA), jnp.int32),
        pltpu.VMEM((CA, 256), jnp.float32),
        pltpu.VMEM((CA, 256), jnp.float32),
        pltpu.VMEM((4, CA), jnp.float32),
    ],
)


def _sums_body(tok_h, dst_h, a_h, z_h, parts_h,
               dstb, abufT, sval, sidx, acc_sh):
    c = lax.axis_index("c")
    s = lax.axis_index("s")
    w = c * 16 + s
    ZT = NPAD * 8 // 16
    pltpu.sync_copy(z_h, acc_sh.at[pl.ds(s * ZT, ZT)])
    ones = jnp.zeros((16,), jnp.float32) + 1.0
    for g in range(C // 16):
        sval[4, pl.ds(g * 16, 16)] = ones
    plsc.subcore_barrier()
    base = w * PW

    def _chunk(i, carry):
        off = base + i * C
        pltpu.sync_copy(dst_h.at[pl.ds(off, C)], dstb.at[0])
        for h in range(HEADS):
            pltpu.sync_copy(a_h.at[pl.ds(h * EPAD + off, C)], abufT.at[h])
        for g in range(C // 16):
            dvec = dstb[0, pl.ds(g * 16, 16)]
            d8 = dvec * 8
            for col in range(4):
                sval[col, pl.ds(g * 16, 16)] = abufT[col, pl.ds(g * 16, 16)]
                sidx[col, pl.ds(g * 16, 16)] = d8 + col
            sidx[4, pl.ds(g * 16, 16)] = d8 + 4
        for col in range(5):
            pltpu.sync_copy(sval.at[col], acc_sh.at[sidx.at[col]], add=True)
        return carry
    lax.fori_loop(0, NCH, _chunk, 0)
    plsc.subcore_barrier()
    pltpu.sync_copy(acc_sh.at[pl.ds(s * ZT, ZT)],
                    parts_h.at[c, pl.ds(s * ZT, ZT)])


_sums_call = pl.kernel(
    _sums_body,
    out_type=jax.ShapeDtypeStruct((2, NPAD * 8), jnp.float32),
    mesh=_mesh,
    scratch_types=[
        pltpu.VMEM((1, C), jnp.int32),
        pltpu.VMEM((4, C), jnp.float32),
        pltpu.VMEM((5, C), jnp.float32),
        pltpu.VMEM((5, C), jnp.int32),
        pltpu.VMEM_SHARED((NPAD * 8,), jnp.float32),
    ],
)


def _dens_body(tok_h, dst_h, a_h, cm_h, z_h, exout_h, parts_h,
               dstb, abufT, exbufT, cvals, sidx, acc_sh):
    c = lax.axis_index("c")
    s = lax.axis_index("s")
    w = c * 16 + s
    ZT = NPAD * 8 // 16
    pltpu.sync_copy(z_h, acc_sh.at[pl.ds(s * ZT, ZT)])
    plsc.subcore_barrier()
    base = w * PW

    def _chunk(i, carry):
        off = base + i * C
        pltpu.sync_copy(dst_h.at[pl.ds(off, C)], dstb.at[0])
        for h in range(HEADS):
            pltpu.sync_copy(a_h.at[pl.ds(h * EPAD + off, C)], abufT.at[h])
        for g in range(C // 16):
            dvec = dstb[0, pl.ds(g * 16, 16)]
            d8 = dvec * 8
            for col in range(4):
                sidx[col, pl.ds(g * 16, 16)] = d8 + col
        for col in range(4):
            pltpu.sync_copy(cm_h.at[sidx.at[col]], cvals.at[col])
        for g in range(C // 16):
            for col in range(4):
                av = abufT[col, pl.ds(g * 16, 16)]
                cv = cvals[col, pl.ds(g * 16, 16)]
                exbufT[col, pl.ds(g * 16, 16)] = jnp.exp(av - cv)
        for col in range(4):
            pltpu.sync_copy(exbufT.at[col], acc_sh.at[sidx.at[col]], add=True)
            pltpu.sync_copy(exbufT.at[col],
                            exout_h.at[pl.ds(col * EPAD + off, C)])
        return carry
    lax.fori_loop(0, NCH, _chunk, 0)
    plsc.subcore_barrier()
    pltpu.sync_copy(acc_sh.at[pl.ds(s * ZT, ZT)],
                    parts_h.at[c, pl.ds(s * ZT, ZT)])


_dens_call = pl.kernel(
    _dens_body,
    out_type=[jax.ShapeDtypeStruct((4 * EPAD,), jnp.float32),
              jax.ShapeDtypeStruct((2, NPAD * 8), jnp.float32)],
    mesh=_mesh,
    scratch_types=[
        pltpu.VMEM((1, C), jnp.int32),
        pltpu.VMEM((4, C), jnp.float32),
        pltpu.VMEM((4, C), jnp.float32),
        pltpu.VMEM((4, C), jnp.float32),
        pltpu.VMEM((4, C), jnp.int32),
        pltpu.VMEM_SHARED((NPAD * 8,), jnp.float32),
    ],
)


def _agg_body(tok_h, dst_h, src_h, ex_h, r_h, vh0_h, vh1_h, z_h, u_h,
              dstb, srcb, vbuf, exbufT, rvals, ribuf, sval, sidx, acc_sh):
    c = lax.axis_index("c")
    s = lax.axis_index("s")
    lane = _lanes()
    consts = [lane * 0 + t for t in range(16)]
    ZT = NPAD * 32 // 16
    pltpu.sync_copy(z_h, acc_sh.at[pl.ds(s * ZT, ZT)])
    plsc.subcore_barrier()
    base = s * PT

    def _chunk(i, carry):
        off = base + i * C
        pltpu.sync_copy(dst_h.at[pl.ds(off, C)], dstb.at[0])
        pltpu.sync_copy(src_h.at[pl.ds(off, C)], srcb.at[0])

        @pl.when(c == 0)
        def _():
            pltpu.sync_copy(vh0_h.at[srcb.at[0]], vbuf)

        @pl.when(c != 0)
        def _():
            pltpu.sync_copy(vh1_h.at[srcb.at[0]], vbuf)

        for h in range(HEADS):
            pltpu.sync_copy(ex_h.at[pl.ds(h * EPAD + off, C)], exbufT.at[h])
        for g in range(C // 16):
            dvec = dstb[0, pl.ds(g * 16, 16)]
            d8 = dvec * 8
            for col in range(4):
                ribuf[col, pl.ds(g * 16, 16)] = d8 + col
        for col in range(4):
            pltpu.sync_copy(r_h.at[ribuf.at[col]], rvals.at[col])
        for g in range(C // 16):
            dvec = dstb[0, pl.ds(g * 16, 16)]
            d32 = dvec * 32
            wcols = [exbufT[h, pl.ds(g * 16, 16)] * rvals[h, pl.ds(g * 16, 16)]
                     for h in range(HEADS)]
            for t in range(16):
                e = g * 16 + t
                wb = [jnp.take_along_axis(wcols[h], consts[t], axis=0)
                      for h in range(HEADS)]
                dstbc = jnp.take_along_axis(d32, consts[t], axis=0)
                row = g * 4 + t // 4
                cbase = (t % 4) * 32
                for half in range(2):
                    m = wb[0] * vbuf[e, pl.ds(half * 16, 16)]
                    m = m + wb[1] * vbuf[e, pl.ds(32 + half * 16, 16)]
                    m = m + wb[2] * vbuf[e, pl.ds(64 + half * 16, 16)]
                    m = m + wb[3] * vbuf[e, pl.ds(96 + half * 16, 16)]
                    sval[row, pl.ds(cbase + half * 16, 16)] = m
                    sidx[row, pl.ds(cbase + half * 16, 16)] = (
                        dstbc + (lane + half * 16))
        for j in range(32):
            pltpu.sync_copy(sval.at[j], acc_sh.at[sidx.at[j]], add=True)
        return carry
    lax.fori_loop(0, NCHA, _chunk, 0)
    plsc.subcore_barrier()
    pltpu.sync_copy(acc_sh.at[pl.ds(s * ZT, ZT)],
                    u_h.at[c, pl.ds(s * ZT, ZT)])


_agg_call = pl.kernel(
    _agg_body,
    out_type=jax.ShapeDtypeStruct((2, NPAD * 32), jnp.float32),
    mesh=_mesh,
    scratch_types=[
        pltpu.VMEM((1, C), jnp.int32),
        pltpu.VMEM((1, C), jnp.int32),
        pltpu.VMEM((C, 128), jnp.float32),
        pltpu.VMEM((4, C), jnp.float32),
        pltpu.VMEM((4, C), jnp.float32),
        pltpu.VMEM((4, C), jnp.int32),
        pltpu.VMEM((32, C), jnp.float32),
        pltpu.VMEM((32, C), jnp.int32),
        pltpu.VMEM_SHARED((NPAD * 32,), jnp.float32),
    ],
)


def _fuse_body(pf, cf, tf, df, wn, bn, wb, bb, wt, bt, wd, bd, w1, b1, o_ref):
    un = _lrelu(jnp.dot(pf[...], wn[...], preferred_element_type=jnp.float32) + bn[...])
    ub = _lrelu(jnp.dot(cf[...], wb[...], preferred_element_type=jnp.float32) + bb[...])
    ut = _lrelu(jnp.dot(tf[...], wt[...], preferred_element_type=jnp.float32) + bt[...])
    ud = _lrelu(jnp.dot(df[...], wd[...], preferred_element_type=jnp.float32) + bd[...])
    x = jnp.concatenate([un, ub, ut, ud], axis=1)
    o_ref[...] = _lrelu(jnp.dot(x, w1[...], preferred_element_type=jnp.float32) + b1[...])


def _nb_spec(d):
    return pl.BlockSpec((_NB, d), lambda i: (i, 0))


def _full_spec(a):
    return pl.BlockSpec(a.shape, lambda i: (0,) * a.ndim)


def _row(b):
    return b.reshape(1, -1)


def _fuse(pf, cf, tf, df, p):
    args = [pf, cf, tf, df,
            p['in_num']['W'], _row(p['in_num']['b']),
            p['in_bool']['W'], _row(p['in_bool']['b']),
            p['in_tweet']['W'], _row(p['in_tweet']['b']),
            p['in_des']['W'], _row(p['in_des']['b']),
            p['lin1']['W'], _row(p['lin1']['b'])]
    specs = [_nb_spec(pf.shape[1]), _nb_spec(cf.shape[1]),
             _nb_spec(tf.shape[1]), _nb_spec(df.shape[1])]
    specs += [_full_spec(a) for a in args[4:]]
    return pl.pallas_call(
        _fuse_body,
        grid=(_GRID,),
        in_specs=specs,
        out_specs=_nb_spec(64),
        out_shape=jax.ShapeDtypeStruct((NPAD, 64), jnp.float32),
    )(*args)


def _proj_body(x, wq, bq, wk, bk, wv0, bv0, wv1, bv1, ws, bs,
               qs_ref, k_ref, v0_ref, v1_ref, s_ref):
    xb = x[...]
    qs_ref[...] = jnp.dot(xb, wq[...], preferred_element_type=jnp.float32) + bq[...]
    k_ref[...] = jnp.dot(xb, wk[...], preferred_element_type=jnp.float32) + bk[...]
    v0_ref[...] = jnp.dot(xb, wv0[...], preferred_element_type=jnp.float32) + bv0[...]
    v1_ref[...] = jnp.dot(xb, wv1[...], preferred_element_type=jnp.float32) + bv1[...]
    s_ref[...] = jnp.dot(xb, ws[...], preferred_element_type=jnp.float32) + bs[...]


def _proj(x, tp):
    wq = tp['q']['W'] * 0.125
    bq = tp['q']['b'] * 0.125
    wv = tp['v']['W']
    bv = tp['v']['b']
    perms = [jnp.asarray([h * 64 + cc * 32 + f for h in range(HEADS)
                          for f in range(32)], dtype=jnp.int32)
             for cc in range(2)]
    args = [x, wq, _row(bq), tp['k']['W'], _row(tp['k']['b']),
            wv[:, perms[0]], _row(bv[perms[0]]),
            wv[:, perms[1]], _row(bv[perms[1]]),
            tp['s']['W'], _row(tp['s']['b'])]
    specs = [_nb_spec(64)] + [_full_spec(a) for a in args[1:]]
    return pl.pallas_call(
        _proj_body,
        grid=(_GRID,),
        in_specs=specs,
        out_specs=[_nb_spec(256), _nb_spec(256), _nb_spec(128),
                   _nb_spec(128), _nb_spec(64)],
        out_shape=[jax.ShapeDtypeStruct((NPAD, 256), jnp.float32),
                   jax.ShapeDtypeStruct((NPAD, 256), jnp.float32),
                   jax.ShapeDtypeStruct((NPAD, 128), jnp.float32),
                   jax.ShapeDtypeStruct((NPAD, 128), jnp.float32),
                   jax.ShapeDtypeStruct((NPAD, 64), jnp.float32)],
    )(*args)


def _meanc_body(p_ref, o_ref):
    sums = p_ref[0] + p_ref[1]
    deg = jnp.maximum(sums[:, 4:5], 1.0)
    o_ref[...] = sums / deg


def _meanc(parts):
    return pl.pallas_call(
        _meanc_body,
        grid=(_GRID,),
        in_specs=[pl.BlockSpec((2, _NB, 8), lambda i: (0, i, 0))],
        out_specs=pl.BlockSpec((_NB, 8), lambda i: (i, 0)),
        out_shape=jax.ShapeDtypeStruct((NPAD, 8), jnp.float32),
    )(parts)


def _recip_body(p_ref, o_ref):
    o_ref[...] = 0.25 / (p_ref[0] + p_ref[1] + 1e-16)


def _recip(parts):
    return pl.pallas_call(
        _recip_body,
        grid=(_GRID,),
        in_specs=[pl.BlockSpec((2, _NB, 8), lambda i: (0, i, 0))],
        out_specs=pl.BlockSpec((_NB, 8), lambda i: (i, 0)),
        out_shape=jax.ShapeDtypeStruct((NPAD, 8), jnp.float32),
    )(parts)


def _gatesem_body(ue0, s0, ue1, s1, x, wg, bg, l1, b1, l2p,
                  z0_ref, z1_ref, ws_ref):
    i = pl.program_id(0)
    xb = x[...]
    rid = i * _NB + lax.broadcasted_iota(jnp.int32, (_NB, 128), 0)
    valid = rid < N
    sums = []
    for (ue, sref, zref) in ((ue0, s0, z0_ref), (ue1, s1, z1_ref)):
        u = jnp.concatenate([ue[0], ue[1]], axis=1) + sref[...]
        pre = jnp.dot(jnp.concatenate([u, xb], axis=1), wg[...],
                      preferred_element_type=jnp.float32) + bg[...]
        a = 1.0 / (1.0 + jnp.exp(-pre))
        z = jnp.tanh(u) * a
        zref[...] = z
        t = jnp.tanh(jnp.dot(z, l1[...], preferred_element_type=jnp.float32) + b1[...])
        wv = jnp.dot(t, l2p[...], preferred_element_type=jnp.float32)
        wv = jnp.where(valid, wv, 0.0)
        sums.append(jnp.sum(wv, axis=0, keepdims=True))
    contrib = jnp.concatenate([sums[0], sums[1],
                               jnp.zeros((6, 128), jnp.float32)], axis=0)

    @pl.when(i == 0)
    def _():
        ws_ref[...] = contrib

    @pl.when(i != 0)
    def _():
        ws_ref[...] = ws_ref[...] + contrib


def _gatesem(u20, s0, u21, s1, x, lp):
    l1 = jnp.concatenate([lp['sem'][h]['l1']['W'] for h in range(4)], axis=1)
    b1 = jnp.concatenate([lp['sem'][h]['l1']['b'] for h in range(4)])
    l2p = jnp.zeros((512, 128), jnp.float32)
    for h in range(4):
        l2p = l2p.at[h * 128:(h + 1) * 128, h].set(lp['sem'][h]['l2']['W'][:, 0])
    args = [u20, s0, u21, s1, x, lp['gate']['W'], _row(lp['gate']['b']), l1, _row(b1), l2p]
    u_spec = pl.BlockSpec((2, _NB, 32), lambda i: (0, i, 0))
    specs = [u_spec, _nb_spec(64), u_spec, _nb_spec(64),
             _nb_spec(64)] + [_full_spec(a) for a in args[5:]]
    return pl.pallas_call(
        _gatesem_body,
        grid=(_GRID,),
        in_specs=specs,
        out_specs=[_nb_spec(64), _nb_spec(64),
                   pl.BlockSpec((8, 128), lambda i: (0, 0))],
        out_shape=[jax.ShapeDtypeStruct((NPAD, 64), jnp.float32),
                   jax.ShapeDtypeStruct((NPAD, 64), jnp.float32),
                   jax.ShapeDtypeStruct((8, 128), jnp.float32)],
    )(*args)


def _xcomb_body(z0, z1, cref, o_ref):
    c0 = cref[0, 0]
    c1 = cref[0, 1]
    o_ref[...] = _lrelu(c0 * z0[...] + c1 * z1[...])


def _xcomb(z0, z1, coef):
    return pl.pallas_call(
        _xcomb_body,
        grid=(_GRID,),
        in_specs=[_nb_spec(64), _nb_spec(64),
                  pl.BlockSpec(memory_space=pltpu.SMEM)],
        out_specs=_nb_spec(64),
        out_shape=jax.ShapeDtypeStruct((NPAD, 64), jnp.float32),
    )(z0, z1, coef)


def _head_body(z0, z1, cref, w1, b1, w2, b2, o_ref):
    c0 = cref[0, 0]
    c1 = cref[0, 1]
    x = _lrelu(c0 * z0[...] + c1 * z1[...])
    h = _lrelu(jnp.dot(x, w1[...], preferred_element_type=jnp.float32) + b1[...])
    o_ref[...] = jnp.dot(h, w2[...], preferred_element_type=jnp.float32) + b2[...]


def _head(z0, z1, coef, p):
    w2 = jnp.pad(p['out2']['W'], ((0, 0), (0, 126)))
    b2 = jnp.pad(p['out2']['b'], (0, 126))
    args = [z0, z1, coef, p['out1']['W'], _row(p['out1']['b']), w2, _row(b2)]
    specs = [_nb_spec(64), _nb_spec(64), pl.BlockSpec(memory_space=pltpu.SMEM)]
    specs += [_full_spec(a) for a in args[3:]]
    return pl.pallas_call(
        _head_body,
        grid=(_GRID,),
        in_specs=specs,
        out_specs=_nb_spec(128),
        out_shape=jax.ShapeDtypeStruct((NPAD, 128), jnp.float32),
    )(*args)


def kernel(prop_features, cat_features, tweet_features, des_features,
           following_edge_index, follows_edge_index, params):
    p = params
    x = _fuse(prop_features, cat_features, tweet_features, des_features, p)

    pad = jnp.full((EPAD - E,), NPAD - 1, dtype=jnp.int32)
    edges = []
    for ei in (following_edge_index, follows_edge_index):
        edges.append((jnp.concatenate([ei[1], pad]),
                      jnp.concatenate([ei[0], pad])))
    z8f = jnp.zeros((NPAD * 8 // 16,), jnp.float32)
    z32f = jnp.zeros((NPAD * 32 // 16,), jnp.float32)
    tok = jnp.zeros((8,), jnp.float32)

    coef = None
    z0 = z1 = None
    for l in range(2):
        lp = p['rgt%d' % l]
        us = []
        ss = []
        for e in range(2):
            tp = lp['tc%d' % e]
            qs, k, vh0, vh1, sproj = _proj(x, tp)
            dst, src = edges[e]
            aout = _alpha_call(tok, dst, src, qs, k)
            tok = aout[:8]
            aparts = _sums_call(tok, dst, aout, z8f)
            tok = aparts[0, :8]
            cmean = _meanc(aparts.reshape(2, NPAD, 8))
            exout, dparts = _dens_call(tok, dst, aout, cmean.reshape(-1), z8f)
            tok = dparts[0, :8]
            rec = _recip(dparts.reshape(2, NPAD, 8))
            u2 = _agg_call(tok, dst, src, exout, rec.reshape(-1),
                           vh0, vh1, z32f)
            tok = u2[0, :8]
            us.append(u2.reshape(2, NPAD, 32))
            ss.append(sproj)
        z0, z1, wsum = _gatesem(us[0], ss[0], us[1], ss[1], x, lp)
        wm = wsum[0:2, 0:4] / float(N)
        beta = jax.nn.softmax(wm, axis=0)
        coef = (jnp.sum(beta, axis=1) / 4.0).reshape(1, 2)
        if l == 0:
            x = _xcomb(z0, z1, coef)

    out = _head(z0, z1, coef, p)
    return out[:N, :2]


# speedup vs baseline: 13.4303x; 1.4843x over previous
import jax
import jax.numpy as jnp
from jax import lax
from jax.experimental import pallas as pl
from jax.experimental.pallas import tpu as pltpu
from jax.experimental.pallas import tpu_sc as plsc

N = 50000
NPAD = 51200
E = 800000
EPAD = 802816
C = 128
NW = 32
PW = EPAD // NW
NCH = PW // C
PT = EPAD // 16
NCHA = PT // C
HEADS = 4

_NB = 2048
_GRID = NPAD // _NB

_mesh = plsc.VectorSubcoreMesh(core_axis_name="c", subcore_axis_name="s",
                               num_cores=2, num_subcores=16)


def _lrelu(t):
    return jnp.where(t >= 0, t, 0.01 * t)


def _lanes():
    return lax.broadcasted_iota(jnp.int32, (16,), 0)


CA = 64
NCHQ = PW // CA


def _alpha_body(tok_h, dst_h, src_h, qs_h, k_h, aout_h,
                ib, qb0, kb0, qb1, kb1, ab0, ab1, sg0, sg1, so):
    c = lax.axis_index("c")
    s = lax.axis_index("s")
    w = c * 16 + s
    lane = _lanes()
    lmasks = [jnp.where(lane == t, 1.0, 0.0) for t in range(16)]
    rots = [(lane + r) % 16 for r in (8, 4, 2, 1)]
    base = w * PW
    pltpu.sync_copy(dst_h.at[pl.ds(base, CA)], ib.at[0, 0])
    pltpu.sync_copy(src_h.at[pl.ds(base, CA)], ib.at[0, 1])
    pltpu.async_copy(qs_h.at[ib.at[0, 0]], qb0, sg0)
    pltpu.async_copy(k_h.at[ib.at[0, 1]], kb0, sg0)

    def _chunk2(gi, carry):
        for slot in (0, 1):
            i = gi * 2 + slot
            qb, kb, ab, sg = ((qb0, kb0, ab0, sg0) if slot == 0
                              else (qb1, kb1, ab1, sg1))
            qbn, kbn, sgn = ((qb1, kb1, sg1) if slot == 0
                             else (qb0, kb0, sg0))
            oslot = 1 - slot

            @pl.when(i + 1 < NCHQ)
            def _():
                noff = base + (i + 1) * CA
                pltpu.sync_copy(dst_h.at[pl.ds(noff, CA)], ib.at[oslot, 0])
                pltpu.sync_copy(src_h.at[pl.ds(noff, CA)], ib.at[oslot, 1])
                pltpu.async_copy(qs_h.at[ib.at[oslot, 0]], qbn, sgn)
                pltpu.async_copy(k_h.at[ib.at[oslot, 1]], kbn, sgn)

            pltpu.make_async_copy(qs_h.at[ib.at[slot, 0]], qb, sg).wait()
            pltpu.make_async_copy(k_h.at[ib.at[slot, 1]], kb, sg).wait()

            @pl.when(i >= 2)
            def _():
                for h in range(HEADS):
                    pltpu.make_async_copy(
                        ab.at[h], aout_h.at[pl.ds(0, CA)], so).wait()
            for g in range(CA // 16):
                cols = [jnp.zeros((16,), jnp.float32) for _ in range(HEADS)]
                for t in range(16):
                    e = g * 16 + t
                    for h in range(HEADS):
                        acc = qb[e, pl.ds(h * 64, 16)] * kb[e, pl.ds(h * 64, 16)]
                        for j in range(1, 4):
                            acc = acc + (qb[e, pl.ds(h * 64 + j * 16, 16)]
                                         * kb[e, pl.ds(h * 64 + j * 16, 16)])
                        for ridx in rots:
                            acc = acc + jnp.take_along_axis(acc, ridx, axis=0)
                        cols[h] = cols[h] + acc * lmasks[t]
                for h in range(HEADS):
                    ab[h, pl.ds(g * 16, 16)] = cols[h]
            off = base + i * CA
            for h in range(HEADS):
                pltpu.async_copy(ab.at[h],
                                 aout_h.at[pl.ds(h * EPAD + off, CA)], so)
        return carry
    lax.fori_loop(0, NCHQ // 2, _chunk2, 0)
    for ab in (ab0, ab1):
        for h in range(HEADS):
            pltpu.make_async_copy(ab.at[h], aout_h.at[pl.ds(0, CA)], so).wait()


_alpha_call = pl.kernel(
    _alpha_body,
    out_type=jax.ShapeDtypeStruct((4 * EPAD,), jnp.float32),
    mesh=_mesh,
    scratch_types=[
        pltpu.VMEM((2, 2, CA), jnp.int32),
        pltpu.VMEM((CA, 256), jnp.float32),
        pltpu.VMEM((CA, 256), jnp.float32),
        pltpu.VMEM((CA, 256), jnp.float32),
        pltpu.VMEM((CA, 256), jnp.float32),
        pltpu.VMEM((4, CA), jnp.float32),
        pltpu.VMEM((4, CA), jnp.float32),
        pltpu.SemaphoreType.DMA,
        pltpu.SemaphoreType.DMA,
        pltpu.SemaphoreType.DMA,
    ],
)


def _sums_body(tok_h, dst_h, a_h, z_h, parts_h,
               dstb, abufT, sval, sidx, acc_sh):
    c = lax.axis_index("c")
    s = lax.axis_index("s")
    w = c * 16 + s
    ZT = NPAD * 5 // 16
    pltpu.sync_copy(z_h, acc_sh.at[pl.ds(s * ZT, ZT)])
    ones = jnp.zeros((16,), jnp.float32) + 1.0
    for g in range(C // 16):
        sval[4, pl.ds(g * 16, 16)] = ones
    plsc.subcore_barrier()
    base = w * PW

    def _chunk(i, carry):
        off = base + i * C
        pltpu.sync_copy(dst_h.at[pl.ds(off, C)], dstb.at[0])
        for h in range(HEADS):
            pltpu.sync_copy(a_h.at[pl.ds(h * EPAD + off, C)], abufT.at[h])
        for g in range(C // 16):
            dvec = dstb[0, pl.ds(g * 16, 16)]
            d5 = dvec * 5
            for col in range(4):
                sval[col, pl.ds(g * 16, 16)] = abufT[col, pl.ds(g * 16, 16)]
                sidx[col, pl.ds(g * 16, 16)] = d5 + col
            sidx[4, pl.ds(g * 16, 16)] = d5 + 4
        for col in range(5):
            pltpu.sync_copy(sval.at[col], acc_sh.at[sidx.at[col]], add=True)
        return carry
    lax.fori_loop(0, NCH, _chunk, 0)
    plsc.subcore_barrier()
    pltpu.sync_copy(acc_sh.at[pl.ds(s * ZT, ZT)],
                    parts_h.at[c, pl.ds(s * ZT, ZT)])


_sums_call = pl.kernel(
    _sums_body,
    out_type=jax.ShapeDtypeStruct((2, NPAD * 5), jnp.float32),
    mesh=_mesh,
    scratch_types=[
        pltpu.VMEM((1, C), jnp.int32),
        pltpu.VMEM((4, C), jnp.float32),
        pltpu.VMEM((5, C), jnp.float32),
        pltpu.VMEM((5, C), jnp.int32),
        pltpu.VMEM_SHARED((NPAD * 5,), jnp.float32),
    ],
)


def _dens_body(tok_h, dst_h, a_h, cm_h, z_h, exout_h, parts_h,
               dstb, abufT, exbufT, cvals, cidx, sidx, acc_sh):
    c = lax.axis_index("c")
    s = lax.axis_index("s")
    w = c * 16 + s
    ZT = NPAD * 4 // 16
    pltpu.sync_copy(z_h, acc_sh.at[pl.ds(s * ZT, ZT)])
    plsc.subcore_barrier()
    base = w * PW

    def _chunk(i, carry):
        off = base + i * C
        pltpu.sync_copy(dst_h.at[pl.ds(off, C)], dstb.at[0])
        for h in range(HEADS):
            pltpu.sync_copy(a_h.at[pl.ds(h * EPAD + off, C)], abufT.at[h])
        for g in range(C // 16):
            dvec = dstb[0, pl.ds(g * 16, 16)]
            d5 = dvec * 5
            d4 = dvec * 4
            for col in range(4):
                cidx[col, pl.ds(g * 16, 16)] = d5 + col
                sidx[col, pl.ds(g * 16, 16)] = d4 + col
        for col in range(4):
            pltpu.sync_copy(cm_h.at[cidx.at[col]], cvals.at[col])
        for g in range(C // 16):
            for col in range(4):
                av = abufT[col, pl.ds(g * 16, 16)]
                cv = cvals[col, pl.ds(g * 16, 16)]
                exbufT[col, pl.ds(g * 16, 16)] = jnp.exp(av - cv)
        for col in range(4):
            pltpu.sync_copy(exbufT.at[col], acc_sh.at[sidx.at[col]], add=True)
            pltpu.sync_copy(exbufT.at[col],
                            exout_h.at[pl.ds(col * EPAD + off, C)])
        return carry
    lax.fori_loop(0, NCH, _chunk, 0)
    plsc.subcore_barrier()
    pltpu.sync_copy(acc_sh.at[pl.ds(s * ZT, ZT)],
                    parts_h.at[c, pl.ds(s * ZT, ZT)])


_dens_call = pl.kernel(
    _dens_body,
    out_type=[jax.ShapeDtypeStruct((4 * EPAD,), jnp.float32),
              jax.ShapeDtypeStruct((2, NPAD * 4), jnp.float32)],
    mesh=_mesh,
    scratch_types=[
        pltpu.VMEM((1, C), jnp.int32),
        pltpu.VMEM((4, C), jnp.float32),
        pltpu.VMEM((4, C), jnp.float32),
        pltpu.VMEM((4, C), jnp.float32),
        pltpu.VMEM((4, C), jnp.int32),
        pltpu.VMEM((4, C), jnp.int32),
        pltpu.VMEM_SHARED((NPAD * 4,), jnp.float32),
    ],
)


def _agg_body(tok_h, dst_h, src_h, ex_h, r_h, vh0_h, vh1_h, z_h, u_h,
              ib, vb0, vb1, exb0, exb1, rv0, rv1, ri0, ri1,
              sval, sidx, acc_sh, sg0, sg1, ss):
    c = lax.axis_index("c")
    s = lax.axis_index("s")
    lane = _lanes()
    consts = [lane * 0 + t for t in range(16)]
    ZT = NPAD
    base = s * PT

    def _prefetch(i, islot, vb, exb, rv, ri, sg):
        off = base + i * C
        pltpu.sync_copy(dst_h.at[pl.ds(off, C)], ib.at[islot, 0])
        pltpu.sync_copy(src_h.at[pl.ds(off, C)], ib.at[islot, 1])

        @pl.when(c == 0)
        def _():
            pltpu.async_copy(vh0_h.at[ib.at[islot, 1]], vb, sg)

        @pl.when(c != 0)
        def _():
            pltpu.async_copy(vh1_h.at[ib.at[islot, 1]], vb, sg)

        for h in range(HEADS):
            pltpu.async_copy(ex_h.at[pl.ds(h * EPAD + off, C)],
                             exb.at[h], sg)
        for g in range(C // 16):
            dvec = ib[islot, 0, pl.ds(g * 16, 16)]
            d4 = dvec * 4
            for col in range(4):
                ri[col, pl.ds(g * 16, 16)] = d4 + col
        for col in range(4):
            pltpu.async_copy(r_h.at[ri.at[col]], rv.at[col], sg)

    for p in range(2):
        pltpu.sync_copy(z_h, acc_sh.at[pl.ds(s * ZT, ZT)])
        plsc.subcore_barrier()
        _prefetch(0, 0, vb0, exb0, rv0, ri0, sg0)

        def _chunk2(gi, carry):
            for slot in (0, 1):
                i = gi * 2 + slot
                vb, exb, rv, ri, sg = ((vb0, exb0, rv0, ri0, sg0)
                                       if slot == 0
                                       else (vb1, exb1, rv1, ri1, sg1))
                vbn, exbn, rvn, rin, sgn = ((vb1, exb1, rv1, ri1, sg1)
                                            if slot == 0
                                            else (vb0, exb0, rv0, ri0, sg0))
                oslot = 1 - slot

                @pl.when(i + 1 < NCHA)
                def _():
                    _prefetch(i + 1, oslot, vbn, exbn, rvn, rin, sgn)

                @pl.when(c == 0)
                def _():
                    pltpu.make_async_copy(vh0_h.at[ib.at[slot, 1]],
                                          vb, sg).wait()

                @pl.when(c != 0)
                def _():
                    pltpu.make_async_copy(vh1_h.at[ib.at[slot, 1]],
                                          vb, sg).wait()
                for h in range(HEADS):
                    pltpu.make_async_copy(ex_h.at[pl.ds(h * EPAD, C)],
                                          exb.at[h], sg).wait()
                for col in range(4):
                    pltpu.make_async_copy(r_h.at[ri.at[col]],
                                          rv.at[col], sg).wait()

                for g in range(C // 16):
                    dvec = ib[slot, 0, pl.ds(g * 16, 16)]
                    d16 = dvec * 16
                    wcols = [exb[h, pl.ds(g * 16, 16)]
                             * rv[h, pl.ds(g * 16, 16)]
                             for h in range(HEADS)]
                    for t in range(16):
                        e = g * 16 + t
                        wb = [jnp.take_along_axis(wcols[h], consts[t], axis=0)
                              for h in range(HEADS)]
                        dstbc = jnp.take_along_axis(d16, consts[t], axis=0)
                        row = g * 2 + t // 8
                        cbase = (t % 8) * 16
                        m = wb[0] * vb[e, pl.ds(p * 16, 16)]
                        m = m + wb[1] * vb[e, pl.ds(32 + p * 16, 16)]
                        m = m + wb[2] * vb[e, pl.ds(64 + p * 16, 16)]
                        m = m + wb[3] * vb[e, pl.ds(96 + p * 16, 16)]
                        sval[row, pl.ds(cbase, 16)] = m
                        sidx[row, pl.ds(cbase, 16)] = dstbc + lane
                for j in range(16):
                    pltpu.async_copy(sval.at[j], acc_sh.at[sidx.at[j]], ss,
                                     add=True)
                for j in range(16):
                    pltpu.make_async_copy(sval.at[j], acc_sh.at[sidx.at[j]],
                                          ss).wait()
            return carry
        lax.fori_loop(0, NCHA // 2, _chunk2, 0)
        plsc.subcore_barrier()
        pltpu.sync_copy(acc_sh.at[pl.ds(s * ZT, ZT)],
                        u_h.at[2 * c + p, pl.ds(s * ZT, ZT)])
        plsc.subcore_barrier()


_agg_call = pl.kernel(
    _agg_body,
    out_type=jax.ShapeDtypeStruct((4, NPAD * 16), jnp.float32),
    mesh=_mesh,
    scratch_types=[
        pltpu.VMEM((2, 2, C), jnp.int32),
        pltpu.VMEM((C, 128), jnp.float32),
        pltpu.VMEM((C, 128), jnp.float32),
        pltpu.VMEM((4, C), jnp.float32),
        pltpu.VMEM((4, C), jnp.float32),
        pltpu.VMEM((4, C), jnp.float32),
        pltpu.VMEM((4, C), jnp.float32),
        pltpu.VMEM((4, C), jnp.int32),
        pltpu.VMEM((4, C), jnp.int32),
        pltpu.VMEM((16, C), jnp.float32),
        pltpu.VMEM((16, C), jnp.int32),
        pltpu.VMEM_SHARED((NPAD * 16,), jnp.float32),
        pltpu.SemaphoreType.DMA,
        pltpu.SemaphoreType.DMA,
        pltpu.SemaphoreType.DMA,
    ],
)


def _fuse_body(pf, cf, tf, df, wn, bn, wb, bb, wt, bt, wd, bd, w1, b1, o_ref):
    un = _lrelu(jnp.dot(pf[...], wn[...], preferred_element_type=jnp.float32) + bn[...])
    ub = _lrelu(jnp.dot(cf[...], wb[...], preferred_element_type=jnp.float32) + bb[...])
    ut = _lrelu(jnp.dot(tf[...], wt[...], preferred_element_type=jnp.float32) + bt[...])
    ud = _lrelu(jnp.dot(df[...], wd[...], preferred_element_type=jnp.float32) + bd[...])
    x = jnp.concatenate([un, ub, ut, ud], axis=1)
    o_ref[...] = _lrelu(jnp.dot(x, w1[...], preferred_element_type=jnp.float32) + b1[...])


def _nb_spec(d):
    return pl.BlockSpec((_NB, d), lambda i: (i, 0))


def _full_spec(a):
    return pl.BlockSpec(a.shape, lambda i: (0,) * a.ndim)


def _row(b):
    return b.reshape(1, -1)


def _fuse(pf, cf, tf, df, p):
    args = [pf, cf, tf, df,
            p['in_num']['W'], _row(p['in_num']['b']),
            p['in_bool']['W'], _row(p['in_bool']['b']),
            p['in_tweet']['W'], _row(p['in_tweet']['b']),
            p['in_des']['W'], _row(p['in_des']['b']),
            p['lin1']['W'], _row(p['lin1']['b'])]
    specs = [_nb_spec(pf.shape[1]), _nb_spec(cf.shape[1]),
             _nb_spec(tf.shape[1]), _nb_spec(df.shape[1])]
    specs += [_full_spec(a) for a in args[4:]]
    return pl.pallas_call(
        _fuse_body,
        grid=(_GRID,),
        in_specs=specs,
        out_specs=_nb_spec(64),
        out_shape=jax.ShapeDtypeStruct((NPAD, 64), jnp.float32),
    )(*args)


def _proj_body(x, wq, bq, wk, bk, wv0, bv0, wv1, bv1, ws, bs,
               qs_ref, k_ref, v0_ref, v1_ref, s_ref):
    xb = x[...]
    qs_ref[...] = jnp.dot(xb, wq[...], preferred_element_type=jnp.float32) + bq[...]
    k_ref[...] = jnp.dot(xb, wk[...], preferred_element_type=jnp.float32) + bk[...]
    v0_ref[...] = jnp.dot(xb, wv0[...], preferred_element_type=jnp.float32) + bv0[...]
    v1_ref[...] = jnp.dot(xb, wv1[...], preferred_element_type=jnp.float32) + bv1[...]
    s_ref[...] = jnp.dot(xb, ws[...], preferred_element_type=jnp.float32) + bs[...]


def _proj(x, tp):
    wq = tp['q']['W'] * 0.125
    bq = tp['q']['b'] * 0.125
    wv = tp['v']['W']
    bv = tp['v']['b']
    perms = [jnp.asarray([h * 64 + cc * 32 + f for h in range(HEADS)
                          for f in range(32)], dtype=jnp.int32)
             for cc in range(2)]
    args = [x, wq, _row(bq), tp['k']['W'], _row(tp['k']['b']),
            wv[:, perms[0]], _row(bv[perms[0]]),
            wv[:, perms[1]], _row(bv[perms[1]]),
            tp['s']['W'], _row(tp['s']['b'])]
    specs = [_nb_spec(64)] + [_full_spec(a) for a in args[1:]]
    return pl.pallas_call(
        _proj_body,
        grid=(_GRID,),
        in_specs=specs,
        out_specs=[_nb_spec(256), _nb_spec(256), _nb_spec(128),
                   _nb_spec(128), _nb_spec(64)],
        out_shape=[jax.ShapeDtypeStruct((NPAD, 256), jnp.float32),
                   jax.ShapeDtypeStruct((NPAD, 256), jnp.float32),
                   jax.ShapeDtypeStruct((NPAD, 128), jnp.float32),
                   jax.ShapeDtypeStruct((NPAD, 128), jnp.float32),
                   jax.ShapeDtypeStruct((NPAD, 64), jnp.float32)],
    )(*args)


def _meanc_body(p_ref, o_ref):
    sums = p_ref[0] + p_ref[1]
    deg = jnp.maximum(sums[:, 4:5], 1.0)
    o_ref[...] = sums / deg


def _meanc(parts):
    return pl.pallas_call(
        _meanc_body,
        grid=(_GRID,),
        in_specs=[pl.BlockSpec((2, _NB, 5), lambda i: (0, i, 0))],
        out_specs=pl.BlockSpec((_NB, 5), lambda i: (i, 0)),
        out_shape=jax.ShapeDtypeStruct((NPAD, 5), jnp.float32),
    )(parts)


def _recip_body(p_ref, o_ref):
    o_ref[...] = 0.25 / (p_ref[0] + p_ref[1] + 1e-16)


def _recip(parts):
    return pl.pallas_call(
        _recip_body,
        grid=(_GRID,),
        in_specs=[pl.BlockSpec((2, _NB, 4), lambda i: (0, i, 0))],
        out_specs=pl.BlockSpec((_NB, 4), lambda i: (i, 0)),
        out_shape=jax.ShapeDtypeStruct((NPAD, 4), jnp.float32),
    )(parts)


def _gatesem_body(ue0, s0, ue1, s1, x, wg, bg, l1, b1, l2p,
                  z0_ref, z1_ref, ws_ref):
    i = pl.program_id(0)
    xb = x[...]
    rid = i * _NB + lax.broadcasted_iota(jnp.int32, (_NB, 128), 0)
    valid = rid < N
    sums = []
    for (ue, sref, zref) in ((ue0, s0, z0_ref), (ue1, s1, z1_ref)):
        u = jnp.concatenate([ue[0], ue[1], ue[2], ue[3]], axis=1) + sref[...]
        pre = jnp.dot(jnp.concatenate([u, xb], axis=1), wg[...],
                      preferred_element_type=jnp.float32) + bg[...]
        a = 1.0 / (1.0 + jnp.exp(-pre))
        z = jnp.tanh(u) * a
        zref[...] = z
        t = jnp.tanh(jnp.dot(z, l1[...], preferred_element_type=jnp.float32) + b1[...])
        wv = jnp.dot(t, l2p[...], preferred_element_type=jnp.float32)
        wv = jnp.where(valid, wv, 0.0)
        sums.append(jnp.sum(wv, axis=0, keepdims=True))
    contrib = jnp.concatenate([sums[0], sums[1],
                               jnp.zeros((6, 128), jnp.float32)], axis=0)

    @pl.when(i == 0)
    def _():
        ws_ref[...] = contrib

    @pl.when(i != 0)
    def _():
        ws_ref[...] = ws_ref[...] + contrib


def _gatesem(u20, s0, u21, s1, x, lp):
    l1 = jnp.concatenate([lp['sem'][h]['l1']['W'] for h in range(4)], axis=1)
    b1 = jnp.concatenate([lp['sem'][h]['l1']['b'] for h in range(4)])
    l2p = jnp.zeros((512, 128), jnp.float32)
    for h in range(4):
        l2p = l2p.at[h * 128:(h + 1) * 128, h].set(lp['sem'][h]['l2']['W'][:, 0])
    args = [u20, s0, u21, s1, x, lp['gate']['W'], _row(lp['gate']['b']), l1, _row(b1), l2p]
    u_spec = pl.BlockSpec((4, _NB, 16), lambda i: (0, i, 0))
    specs = [u_spec, _nb_spec(64), u_spec, _nb_spec(64),
             _nb_spec(64)] + [_full_spec(a) for a in args[5:]]
    return pl.pallas_call(
        _gatesem_body,
        grid=(_GRID,),
        in_specs=specs,
        out_specs=[_nb_spec(64), _nb_spec(64),
                   pl.BlockSpec((8, 128), lambda i: (0, 0))],
        out_shape=[jax.ShapeDtypeStruct((NPAD, 64), jnp.float32),
                   jax.ShapeDtypeStruct((NPAD, 64), jnp.float32),
                   jax.ShapeDtypeStruct((8, 128), jnp.float32)],
    )(*args)


def _xcomb_body(z0, z1, cref, o_ref):
    c0 = cref[0, 0]
    c1 = cref[0, 1]
    o_ref[...] = _lrelu(c0 * z0[...] + c1 * z1[...])


def _xcomb(z0, z1, coef):
    return pl.pallas_call(
        _xcomb_body,
        grid=(_GRID,),
        in_specs=[_nb_spec(64), _nb_spec(64),
                  pl.BlockSpec(memory_space=pltpu.SMEM)],
        out_specs=_nb_spec(64),
        out_shape=jax.ShapeDtypeStruct((NPAD, 64), jnp.float32),
    )(z0, z1, coef)


def _head_body(z0, z1, cref, w1, b1, w2, b2, o_ref):
    c0 = cref[0, 0]
    c1 = cref[0, 1]
    x = _lrelu(c0 * z0[...] + c1 * z1[...])
    h = _lrelu(jnp.dot(x, w1[...], preferred_element_type=jnp.float32) + b1[...])
    o_ref[...] = jnp.dot(h, w2[...], preferred_element_type=jnp.float32) + b2[...]


def _head(z0, z1, coef, p):
    w2 = jnp.pad(p['out2']['W'], ((0, 0), (0, 126)))
    b2 = jnp.pad(p['out2']['b'], (0, 126))
    args = [z0, z1, coef, p['out1']['W'], _row(p['out1']['b']), w2, _row(b2)]
    specs = [_nb_spec(64), _nb_spec(64), pl.BlockSpec(memory_space=pltpu.SMEM)]
    specs += [_full_spec(a) for a in args[3:]]
    return pl.pallas_call(
        _head_body,
        grid=(_GRID,),
        in_specs=specs,
        out_specs=_nb_spec(128),
        out_shape=jax.ShapeDtypeStruct((NPAD, 128), jnp.float32),
    )(*args)


def kernel(prop_features, cat_features, tweet_features, des_features,
           following_edge_index, follows_edge_index, params):
    p = params
    x = _fuse(prop_features, cat_features, tweet_features, des_features, p)

    pad = jnp.full((EPAD - E,), NPAD - 1, dtype=jnp.int32)
    edges = []
    for ei in (following_edge_index, follows_edge_index):
        edges.append((jnp.concatenate([ei[1], pad]),
                      jnp.concatenate([ei[0], pad])))
    z5f = jnp.zeros((NPAD * 5 // 16,), jnp.float32)
    z4f = jnp.zeros((NPAD * 4 // 16,), jnp.float32)
    z32f = jnp.zeros((NPAD,), jnp.float32)
    tok = jnp.zeros((8,), jnp.float32)

    coef = None
    z0 = z1 = None
    for l in range(2):
        lp = p['rgt%d' % l]
        us = []
        ss = []
        for e in range(2):
            tp = lp['tc%d' % e]
            qs, k, vh0, vh1, sproj = _proj(x, tp)
            dst, src = edges[e]
            aout = _alpha_call(tok, dst, src, qs, k)
            tok = aout[:8]
            aparts = _sums_call(tok, dst, aout, z5f)
            tok = aparts[0, :8]
            cmean = _meanc(aparts.reshape(2, NPAD, 5))
            exout, dparts = _dens_call(tok, dst, aout, cmean.reshape(-1), z4f)
            tok = dparts[0, :8]
            rec = _recip(dparts.reshape(2, NPAD, 4))
            u2 = _agg_call(tok, dst, src, exout, rec.reshape(-1),
                           vh0, vh1, z32f)
            tok = u2[0, :8]
            us.append(u2.reshape(4, NPAD, 16))
            ss.append(sproj)
        z0, z1, wsum = _gatesem(us[0], ss[0], us[1], ss[1], x, lp)
        wm = wsum[0:2, 0:4] / float(N)
        beta = jax.nn.softmax(wm, axis=0)
        coef = (jnp.sum(beta, axis=1) / 4.0).reshape(1, 2)
        if l == 0:
            x = _xcomb(z0, z1, coef)

    out = _head(z0, z1, coef, p)
    return out[:N, :2]
